# baseline (speedup 1.0000x reference)
import jax
import jax.numpy as jnp
from jax import lax
from jax.experimental import pallas as pl
from jax.experimental.pallas import tpu as pltpu
from jax.experimental.pallas import tpu_sc as plsc

SIZES = 100000
RANK = 128
HALF = RANK // 2
MARGIN = 1.0
NC = 2
NS = 16
NW = NC * NS
L = 16

BLK = 128
NFULL = SIZES // BLK
REM = SIZES - NFULL * BLK


def _transpose_block(src_v, dst_v, nrows, ncols, lanes):
    rots = [jnp.bitwise_and(lanes + k, L - 1) for k in range(L)]

    @plsc.parallel_loop(0, nrows // L, step=1, unroll=2)
    def _(g):
        rowvec = g * L + lanes
        for c0 in range(0, ncols, L):
            for k in range(L):
                colvec = rots[k] + c0
                v = plsc.load_gather(src_v, [rowvec, colvec])
                plsc.store_scatter(dst_v, [colvec, rowvec], v)


def _sc_sweep(e1t, e2t, e1r, e2r, blk0, blk1, out0, out1, rsem, wsem):
    wid = lax.axis_index("s") * NC + lax.axis_index("c")
    lanes = lax.iota(jnp.int32, L)
    nfull = jnp.where(wid <= (NFULL - 1) % NW, NFULL // NW + 1, NFULL // NW)
    blks = (blk0, blk1)
    outs = (out0, out1)

    def run_table(src, dst):
        def start_read(i, b):
            c0 = (i * NW + wid) * BLK
            pltpu.async_copy(src.at[:, pl.ds(c0, BLK)], blks[b], rsem)

        def drain_read():
            pltpu.make_async_copy(
                src.at[:, pl.ds(0, BLK)], blk0, rsem).wait()

        def drain_write():
            pltpu.make_async_copy(out0, dst.at[pl.ds(0, BLK)], wsem).wait()

        @pl.when(nfull > 0)
        def _():
            start_read(0, 0)

        @pl.when(nfull > 1)
        def _():
            start_read(1, 1)

        def body(j, carry):
            for b in range(2):
                i = j * 2 + b

                @pl.when(i < nfull)
                def _():
                    drain_read()

                    @pl.when(i >= 2)
                    def _():
                        drain_write()

                    _transpose_block(blks[b], outs[b], HALF, BLK, lanes)

                    @pl.when(i + 2 < nfull)
                    def _():
                        start_read(i + 2, b)

                    c0 = (i * NW + wid) * BLK
                    pltpu.async_copy(outs[b], dst.at[pl.ds(c0, BLK)], wsem)
            return carry

        lax.fori_loop(0, (nfull + 1) // 2, body, 0)

        @pl.when(nfull > 0)
        def _():
            drain_write()

        @pl.when(nfull > 1)
        def _():
            drain_write()

    run_table(e1t, e1r)
    run_table(e2t, e2r)

    def tail_block(src, dst, c0, width):
        hs = [pltpu.async_copy(src.at[d, pl.ds(c0, width)],
                               blk0.at[d, pl.ds(0, width)], wsem)
              for d in range(HALF)]
        for h in hs:
            h.wait()
        _transpose_block(blk0, out0, HALF, width, lanes)
        hs = [pltpu.async_copy(out0.at[r], dst.at[c0 + r], wsem)
              for r in range(width)]
        for h in hs:
            h.wait()

    tail_jobs = []
    for off in range(0, REM, 128):
        for tab in range(2):
            tail_jobs.append((tab, NFULL * BLK + off, min(128, REM - off)))
    for j, (tab, c0, width) in enumerate(tail_jobs):
        src, dst = (e1t, e1r) if tab == 0 else (e2t, e2r)

        @pl.when(wid == NW - 1 - j)
        def _(src=src, dst=dst, c0=c0, width=width):
            tail_block(src, dst, c0, width)


def _sc_gather(b_per_w):
    def body(idxh_hbm, idxr_hbm, idxt_hbm, idxs_hbm, e0, e1r, e2r,
             pred_out, head_out, relt_out, tail_out, tst_out,
             ih_v, it_v, ir_v, is_v,
             head_v, rel_v, tail_v, ts_v, relt_v, tst_v, sos_v, pred_v,
             gsem, rsem, wsem):
        wid = lax.axis_index("s") * NC + lax.axis_index("c")
        base = wid * b_per_w
        lanes = lax.iota(jnp.int32, L)

        pltpu.sync_copy(idxh_hbm.at[pl.ds(base, b_per_w)], ih_v)
        pltpu.sync_copy(idxt_hbm.at[pl.ds(base, b_per_w)], it_v)
        pltpu.sync_copy(idxr_hbm.at[pl.ds(base, b_per_w)], ir_v)
        pltpu.sync_copy(idxs_hbm.at[pl.ds(base, b_per_w)], is_v)

        g1 = pltpu.async_copy(e0.at[ih_v], head_v, gsem)
        g3 = pltpu.async_copy(e0.at[it_v], tail_v, gsem)

        def rel_dma_body(g, carry):
            rbase = g * L
            vr = ir_v[pl.ds(rbase, L)]
            for j in range(L):
                pltpu.async_copy(e1r.at[vr[j]], rel_v.at[rbase + j], rsem)
            return carry

        def ts_dma_body(g, carry):
            rbase = g * L
            vs = is_v[pl.ds(rbase, L)]
            for j in range(L):
                pltpu.async_copy(e2r.at[vs[j]], ts_v.at[rbase + j], rsem)
            return carry

        lax.fori_loop(0, b_per_w // L, rel_dma_body, 0)
        lax.fori_loop(0, b_per_w // L, ts_dma_body, 0)

        g1.wait()
        g3.wait()
        pltpu.make_async_copy(e1r.at[pl.ds(0, b_per_w)], rel_v, rsem).wait()
        pltpu.make_async_copy(e2r.at[pl.ds(0, b_per_w)], ts_v, rsem).wait()

        w1 = pltpu.async_copy(head_v, head_out.at[pl.ds(base, b_per_w)], wsem)
        w3 = pltpu.async_copy(tail_v, tail_out.at[pl.ds(base, b_per_w)], wsem)

        _transpose_block(rel_v, relt_v, b_per_w, HALF, lanes)
        _transpose_block(ts_v, tst_v, b_per_w, HALF, lanes)

        w2 = pltpu.async_copy(relt_v, relt_out.at[:, pl.ds(base, b_per_w)],
                              wsem)
        w4 = pltpu.async_copy(tst_v, tst_out.at[:, pl.ds(base, b_per_w)],
                              wsem)

        def row_body(r, carry):
            rl = jnp.broadcast_to(r, (L,))
            acc = jnp.zeros((L,), jnp.float32)
            for k in range(RANK // L):
                h = head_v[r, pl.ds(k * L, L)]
                t = tail_v[r, pl.ds(k * L, L)]
                if k < HALF // L:
                    rt = rel_v[r, pl.ds(k * L, L)]
                else:
                    rt = ts_v[r, pl.ds(k * L - HALF, L)]
                d = h + rt - t
                acc = acc + d * d
            plsc.store_scatter(sos_v, [lanes, rl], acc)
            return carry

        lax.fori_loop(0, b_per_w, row_body, 0)

        for g in range(b_per_w // L):
            s = sos_v[0, pl.ds(g * L, L)]
            for l in range(1, L):
                s = s + sos_v[l, pl.ds(g * L, L)]
            sc = jnp.maximum(s, 1e-30)
            i = lax.bitcast_convert_type(sc, jnp.int32)
            i = jnp.int32(0x5F3759DF) - lax.shift_right_arithmetic(i, 1)
            y = lax.bitcast_convert_type(i, jnp.float32)
            for _ in range(4):
                y = y * (1.5 - 0.5 * sc * y * y)
            pred_v[pl.ds(g * L, L)] = MARGIN - s * y

        pltpu.sync_copy(pred_v, pred_out.at[pl.ds(base, b_per_w)])
        w1.wait()
        w2.wait()
        w3.wait()
        w4.wait()

    return body


def kernel(x_data, E0, E1, E2, bh, bt):
    del bh, bt
    B = x_data.shape[0]
    b_per_w = B // NW
    idx_h = x_data[:, 0]
    idx_r = x_data[:, 1]
    idx_t = x_data[:, 2]
    idx_s = x_data[:, 3]

    mesh = plsc.VectorSubcoreMesh(core_axis_name="c", subcore_axis_name="s")
    params = pltpu.CompilerParams(needs_layout_passes=False)

    sweep = pl.kernel(
        _sc_sweep,
        out_type=(
            jax.ShapeDtypeStruct((SIZES, HALF), jnp.float32),
            jax.ShapeDtypeStruct((SIZES, HALF), jnp.float32),
        ),
        mesh=mesh,
        scratch_types=[
            pltpu.VMEM((HALF, BLK), jnp.float32),
            pltpu.VMEM((HALF, BLK), jnp.float32),
            pltpu.VMEM((BLK, HALF), jnp.float32),
            pltpu.VMEM((BLK, HALF), jnp.float32),
            pltpu.SemaphoreType.DMA,
            pltpu.SemaphoreType.DMA,
        ],
        compiler_params=params,
    )
    e1r, e2r = sweep(E1.T, E2.T)

    gather = pl.kernel(
        _sc_gather(b_per_w),
        out_type=(
            jax.ShapeDtypeStruct((B,), jnp.float32),
            jax.ShapeDtypeStruct((B, RANK), jnp.float32),
            jax.ShapeDtypeStruct((HALF, B), jnp.float32),
            jax.ShapeDtypeStruct((B, RANK), jnp.float32),
            jax.ShapeDtypeStruct((HALF, B), jnp.float32),
        ),
        mesh=mesh,
        scratch_types=[
            pltpu.VMEM((b_per_w,), jnp.int32),
            pltpu.VMEM((b_per_w,), jnp.int32),
            pltpu.VMEM((b_per_w,), jnp.int32),
            pltpu.VMEM((b_per_w,), jnp.int32),
            pltpu.VMEM((b_per_w, RANK), jnp.float32),
            pltpu.VMEM((b_per_w, HALF), jnp.float32),
            pltpu.VMEM((b_per_w, RANK), jnp.float32),
            pltpu.VMEM((b_per_w, HALF), jnp.float32),
            pltpu.VMEM((HALF, b_per_w), jnp.float32),
            pltpu.VMEM((HALF, b_per_w), jnp.float32),
            pltpu.VMEM((L, b_per_w + 1), jnp.float32),
            pltpu.VMEM((b_per_w,), jnp.float32),
            pltpu.SemaphoreType.DMA,
            pltpu.SemaphoreType.DMA,
            pltpu.SemaphoreType.DMA,
        ],
        compiler_params=params,
    )
    preds, head_e, relt, tail_e, tst = gather(
        idx_h, idx_r, idx_t, idx_s, E0, e1r, e2r)
    return (preds, (head_e, relt.T, tail_e, tst.T))


# speedup vs baseline: 1.0382x; 1.0382x over previous
import jax
import jax.numpy as jnp
from jax import lax
from jax.experimental import pallas as pl
from jax.experimental.pallas import tpu as pltpu
from jax.experimental.pallas import tpu_sc as plsc

RANK = 128
HALF = RANK // 2
MARGIN = 1.0
NC = 2
NS = 16
NW = NC * NS
L = 16


def _sc_transe(b_per_w):
    def body(idxh_hbm, idxr_hbm, idxt_hbm, idxs_hbm, e0, e1, e2,
             pred_out, head_out, rel_out, tail_out, ts_out,
             ih_v, ir_v, it_v, is_v,
             head_v, rel_v, tail_v, ts_v, sos_v, pred_v,
             gsem, wsem):
        wid = lax.axis_index("s") * NC + lax.axis_index("c")
        base = wid * b_per_w
        lanes = lax.iota(jnp.int32, L)

        pltpu.sync_copy(idxh_hbm.at[pl.ds(base, b_per_w)], ih_v)
        pltpu.sync_copy(idxr_hbm.at[pl.ds(base, b_per_w)], ir_v)
        pltpu.sync_copy(idxt_hbm.at[pl.ds(base, b_per_w)], it_v)
        pltpu.sync_copy(idxs_hbm.at[pl.ds(base, b_per_w)], is_v)

        g1 = pltpu.async_copy(e0.at[ih_v], head_v, gsem)
        g2 = pltpu.async_copy(e1.at[ir_v], rel_v, gsem)
        g3 = pltpu.async_copy(e0.at[it_v], tail_v, gsem)
        g4 = pltpu.async_copy(e2.at[is_v], ts_v, gsem)
        g1.wait()
        g2.wait()
        g3.wait()
        g4.wait()

        w1 = pltpu.async_copy(head_v, head_out.at[pl.ds(base, b_per_w)], wsem)
        w2 = pltpu.async_copy(rel_v, rel_out.at[pl.ds(base, b_per_w)], wsem)
        w3 = pltpu.async_copy(tail_v, tail_out.at[pl.ds(base, b_per_w)], wsem)
        w4 = pltpu.async_copy(ts_v, ts_out.at[pl.ds(base, b_per_w)], wsem)

        def row_body(r, carry):
            rl = jnp.broadcast_to(r, (L,))
            acc = jnp.zeros((L,), jnp.float32)
            for k in range(RANK // L):
                h = head_v[r, pl.ds(k * L, L)]
                t = tail_v[r, pl.ds(k * L, L)]
                if k < HALF // L:
                    rt = rel_v[r, pl.ds(k * L, L)]
                else:
                    rt = ts_v[r, pl.ds(k * L - HALF, L)]
                d = h + rt - t
                acc = acc + d * d
            plsc.store_scatter(sos_v, [lanes, rl], acc)
            return carry

        lax.fori_loop(0, b_per_w, row_body, 0)

        for g in range(b_per_w // L):
            s = sos_v[0, pl.ds(g * L, L)]
            for l in range(1, L):
                s = s + sos_v[l, pl.ds(g * L, L)]
            sc = jnp.maximum(s, 1e-30)
            i = lax.bitcast_convert_type(sc, jnp.int32)
            i = jnp.int32(0x5F3759DF) - lax.shift_right_arithmetic(i, 1)
            y = lax.bitcast_convert_type(i, jnp.float32)
            for _ in range(4):
                y = y * (1.5 - 0.5 * sc * y * y)
            pred_v[pl.ds(g * L, L)] = MARGIN - s * y

        pltpu.sync_copy(pred_v, pred_out.at[pl.ds(base, b_per_w)])
        w1.wait()
        w2.wait()
        w3.wait()
        w4.wait()

    return body


def kernel(x_data, E0, E1, E2, bh, bt):
    del bh, bt
    B = x_data.shape[0]
    b_per_w = B // NW
    idx_h = x_data[:, 0]
    idx_r = x_data[:, 1]
    idx_t = x_data[:, 2]
    idx_s = x_data[:, 3]

    mesh = plsc.VectorSubcoreMesh(core_axis_name="c", subcore_axis_name="s")
    out_type = (
        jax.ShapeDtypeStruct((B,), jnp.float32),
        jax.ShapeDtypeStruct((B, RANK), jnp.float32),
        jax.ShapeDtypeStruct((B, HALF), jnp.float32),
        jax.ShapeDtypeStruct((B, RANK), jnp.float32),
        jax.ShapeDtypeStruct((B, HALF), jnp.float32),
    )
    scratch = [
        pltpu.VMEM((b_per_w,), jnp.int32),
        pltpu.VMEM((b_per_w,), jnp.int32),
        pltpu.VMEM((b_per_w,), jnp.int32),
        pltpu.VMEM((b_per_w,), jnp.int32),
        pltpu.VMEM((b_per_w, RANK), jnp.float32),
        pltpu.VMEM((b_per_w, HALF), jnp.float32),
        pltpu.VMEM((b_per_w, RANK), jnp.float32),
        pltpu.VMEM((b_per_w, HALF), jnp.float32),
        pltpu.VMEM((L, b_per_w + 1), jnp.float32),
        pltpu.VMEM((b_per_w,), jnp.float32),
        pltpu.SemaphoreType.DMA,
        pltpu.SemaphoreType.DMA,
    ]
    fn = pl.kernel(_sc_transe(b_per_w), out_type=out_type, mesh=mesh,
                   scratch_types=scratch,
                   compiler_params=pltpu.CompilerParams(
                       needs_layout_passes=False,
                       use_tc_tiling_on_sc=False))
    preds, head_e, rel_e, tail_e, ts_e = fn(
        idx_h, idx_r, idx_t, idx_s, E0, E1, E2)
    return (preds, (head_e, rel_e, tail_e, ts_e))
